# Initial kernel scaffold; baseline (speedup 1.0000x reference)
import jax
import jax.numpy as jnp
from jax.experimental import pallas as pl


def kernel(token_ids, table):
    raise NotImplementedError("write your pallas kernel here")


# speedup vs baseline: 2.1631x; 2.1631x over previous
import functools

import jax
import jax.numpy as jnp
from jax import lax
from jax.experimental import pallas as pl
from jax.experimental.pallas import tpu as pltpu
from jax.experimental.pallas import tpu_sc as plsc

NC, NS, L = 2, 16, 16
NW = NC * NS
BATCH, HIST, D = 16384, 200, 64
RPW = BATCH // NW
TB = 16
NBLK = RPW // TB
FULL_CHUNKS = HIST // L
TAIL = HIST - FULL_CHUNKS * L


def _sc_body(tok_hbm, table_hbm, out_hbm, tokbuf, rows, outbuf, gsem):
    wid = lax.axis_index("s") * NC + lax.axis_index("c")
    base = wid * RPW

    def process_block(b, _):
        rb = base + b * TB
        pltpu.sync_copy(tok_hbm.at[pl.ds(rb, TB)], tokbuf)

        def process_row(r, _):
            cp1 = pltpu.async_copy(
                table_hbm.at[tokbuf.at[r, pl.ds(0, 128)]],
                rows.at[pl.ds(0, 128)], gsem)
            cp2 = pltpu.async_copy(
                table_hbm.at[tokbuf.at[r, pl.ds(128, HIST - 128)]],
                rows.at[pl.ds(128, HIST - 128)], gsem)
            cp1.wait()
            cp2.wait()

            zeros = jnp.zeros((L,), jnp.float32)
            izeros = jnp.zeros((L,), jnp.int32)

            def chunk(i, carry):
                a0, a1, a2, a3, cnt = carry
                t = tokbuf[r, pl.ds(i * L, L)]
                cnt = cnt + plsc.all_reduce_population_count(t != 0)

                def row16(k, acc):
                    a0, a1, a2, a3 = acc
                    rr = i * L + k
                    a0 = a0 + rows[rr, pl.ds(0, L)]
                    a1 = a1 + rows[rr, pl.ds(L, L)]
                    a2 = a2 + rows[rr, pl.ds(2 * L, L)]
                    a3 = a3 + rows[rr, pl.ds(3 * L, L)]
                    return (a0, a1, a2, a3)

                a0, a1, a2, a3 = lax.fori_loop(0, L, row16, (a0, a1, a2, a3))
                return (a0, a1, a2, a3, cnt)

            a0, a1, a2, a3, cnt = lax.fori_loop(
                0, FULL_CHUNKS, chunk, (zeros, zeros, zeros, zeros, izeros))

            t = tokbuf[r, pl.ds(HIST - L, L)]
            lane = lax.iota(jnp.int32, 16)
            cnt = cnt + plsc.all_reduce_population_count(
                (lane >= L - TAIL) & (t != 0))

            def tailrow(k, acc):
                a0, a1, a2, a3 = acc
                rr = FULL_CHUNKS * L + k
                a0 = a0 + rows[rr, pl.ds(0, L)]
                a1 = a1 + rows[rr, pl.ds(L, L)]
                a2 = a2 + rows[rr, pl.ds(2 * L, L)]
                a3 = a3 + rows[rr, pl.ds(3 * L, L)]
                return (a0, a1, a2, a3)

            a0, a1, a2, a3 = lax.fori_loop(0, TAIL, tailrow, (a0, a1, a2, a3))

            sv = 1.0 / jnp.maximum(cnt.astype(jnp.float32), 1.0)
            ro = b * TB + r
            outbuf[ro, pl.ds(0, L)] = a0 * sv
            outbuf[ro, pl.ds(L, L)] = a1 * sv
            outbuf[ro, pl.ds(2 * L, L)] = a2 * sv
            outbuf[ro, pl.ds(3 * L, L)] = a3 * sv
            return ()

        lax.fori_loop(0, TB, process_row, ())
        return ()

    lax.fori_loop(0, NBLK, process_block, ())
    pltpu.sync_copy(outbuf, out_hbm.at[pl.ds(base, RPW)])


_sc_kernel = functools.partial(
    pl.kernel,
    out_type=jax.ShapeDtypeStruct((BATCH, D), jnp.float32),
    mesh=plsc.VectorSubcoreMesh(
        core_axis_name="c", subcore_axis_name="s",
        num_cores=NC, num_subcores=NS),
    scratch_types=[
        pltpu.VMEM((TB, HIST), jnp.int32),
        pltpu.VMEM((HIST, D), jnp.float32),
        pltpu.VMEM((RPW, D), jnp.float32),
        pltpu.SemaphoreType.DMA,
    ],
    compiler_params=pltpu.CompilerParams(
        needs_layout_passes=False, use_tc_tiling_on_sc=False),
)(_sc_body)


def kernel(token_ids, table):
    return _sc_kernel(token_ids, table)


# speedup vs baseline: 2.8769x; 1.3300x over previous
import functools

import jax
import jax.numpy as jnp
from jax import lax
from jax.experimental import pallas as pl
from jax.experimental.pallas import tpu as pltpu
from jax.experimental.pallas import tpu_sc as plsc

NC, NS, L = 2, 16, 16
NW = NC * NS
BATCH, HIST, D = 16384, 200, 64
RPW = BATCH // NW
TB = 16
NBLK = RPW // TB
NPAIRS = RPW // 2
G0 = 128
G1 = HIST - G0
FULL_CHUNKS = HIST // L
TAIL = HIST - FULL_CHUNKS * L


def _sc_body(tok_hbm, table_hbm, out_hbm, tokbuf, rows, outbuf,
             tsem, gsem0, gsem1):
    wid = lax.axis_index("s") * NC + lax.axis_index("c")
    base = wid * RPW

    def issue_gathers(r_local, tslot, rslot, gsem):
        rl = r_local & (TB - 1)
        pltpu.async_copy(
            table_hbm.at[tokbuf.at[tslot, rl, pl.ds(0, G0)]],
            rows.at[rslot, pl.ds(0, G0)], gsem)
        pltpu.async_copy(
            table_hbm.at[tokbuf.at[tslot, rl, pl.ds(G0, G1)]],
            rows.at[rslot, pl.ds(G0, G1)], gsem)

    def wait_gathers(rslot, gsem):
        pltpu.make_async_copy(
            table_hbm.at[tokbuf.at[0, 0, pl.ds(0, G0)]],
            rows.at[rslot, pl.ds(0, G0)], gsem).wait()
        pltpu.make_async_copy(
            table_hbm.at[tokbuf.at[0, 0, pl.ds(G0, G1)]],
            rows.at[rslot, pl.ds(G0, G1)], gsem).wait()

    def compute_row(r_local, tslot, rslot):
        rl = r_local & (TB - 1)
        zeros = jnp.zeros((L,), jnp.float32)
        izeros = jnp.zeros((L,), jnp.int32)

        def chunk(i, carry):
            a0, a1, a2, a3, cnt = carry
            t = tokbuf[tslot, rl, pl.ds(i * L, L)]
            cnt = cnt + plsc.all_reduce_population_count(t != 0)
            for k in range(L):
                rr = i * L + k
                a0 = a0 + rows[rslot, rr, pl.ds(0, L)]
                a1 = a1 + rows[rslot, rr, pl.ds(L, L)]
                a2 = a2 + rows[rslot, rr, pl.ds(2 * L, L)]
                a3 = a3 + rows[rslot, rr, pl.ds(3 * L, L)]
            return (a0, a1, a2, a3, cnt)

        a0, a1, a2, a3, cnt = lax.fori_loop(
            0, FULL_CHUNKS, chunk, (zeros, zeros, zeros, zeros, izeros))

        t = tokbuf[tslot, rl, pl.ds(HIST - L, L)]
        lane = lax.iota(jnp.int32, 16)
        cnt = cnt + plsc.all_reduce_population_count(
            (lane >= L - TAIL) & (t != 0))
        for k in range(TAIL):
            rr = FULL_CHUNKS * L + k
            a0 = a0 + rows[rslot, rr, pl.ds(0, L)]
            a1 = a1 + rows[rslot, rr, pl.ds(L, L)]
            a2 = a2 + rows[rslot, rr, pl.ds(2 * L, L)]
            a3 = a3 + rows[rslot, rr, pl.ds(3 * L, L)]

        sv = 1.0 / jnp.maximum(cnt.astype(jnp.float32), 1.0)
        outbuf[r_local, pl.ds(0, L)] = a0 * sv
        outbuf[r_local, pl.ds(L, L)] = a1 * sv
        outbuf[r_local, pl.ds(2 * L, L)] = a2 * sv
        outbuf[r_local, pl.ds(3 * L, L)] = a3 * sv

    pltpu.sync_copy(tok_hbm.at[pl.ds(base, TB)], tokbuf.at[0])
    issue_gathers(0, 0, 0, gsem0)

    def pair(p, _):
        b = p // (TB // 2)
        tslot = b & 1
        r0 = 2 * p
        r1 = 2 * p + 1

        @pl.when(jnp.logical_and((p & (TB // 2 - 1)) == 0, b + 1 < NBLK))
        def _():
            pltpu.async_copy(
                tok_hbm.at[pl.ds(base + (b + 1) * TB, TB)],
                tokbuf.at[(b + 1) & 1], tsem)

        wait_gathers(0, gsem0)
        issue_gathers(r1, tslot, 1, gsem1)
        compute_row(r0, tslot, 0)

        @pl.when(jnp.logical_and((p & (TB // 2 - 1)) == TB // 2 - 1,
                                 b + 1 < NBLK))
        def _():
            pltpu.make_async_copy(
                tok_hbm.at[pl.ds(base, TB)], tokbuf.at[0], tsem).wait()

        wait_gathers(1, gsem1)

        @pl.when(p + 1 < NPAIRS)
        def _():
            issue_gathers(r0 + 2, ((r0 + 2) // TB) & 1, 0, gsem0)

        compute_row(r1, tslot, 1)
        return ()

    lax.fori_loop(0, NPAIRS, pair, ())
    pltpu.sync_copy(outbuf, out_hbm.at[pl.ds(base, RPW)])


_sc_kernel = functools.partial(
    pl.kernel,
    out_type=jax.ShapeDtypeStruct((BATCH, D), jnp.float32),
    mesh=plsc.VectorSubcoreMesh(
        core_axis_name="c", subcore_axis_name="s",
        num_cores=NC, num_subcores=NS),
    scratch_types=[
        pltpu.VMEM((2, TB, HIST), jnp.int32),
        pltpu.VMEM((2, HIST, D), jnp.float32),
        pltpu.VMEM((RPW, D), jnp.float32),
        pltpu.SemaphoreType.DMA,
        pltpu.SemaphoreType.DMA,
        pltpu.SemaphoreType.DMA,
    ],
    compiler_params=pltpu.CompilerParams(
        needs_layout_passes=False, use_tc_tiling_on_sc=False),
)(_sc_body)


def kernel(token_ids, table):
    return _sc_kernel(token_ids, table)


# speedup vs baseline: 3.8774x; 1.3478x over previous
import functools

import jax
import jax.numpy as jnp
from jax import lax
from jax.experimental import pallas as pl
from jax.experimental.pallas import tpu as pltpu
from jax.experimental.pallas import tpu_sc as plsc

NC, NS, L = 2, 16, 16
NW = NC * NS
BATCH, HIST, D = 16384, 200, 64
RPW = BATCH // NW
TB = 16
NBLK = RPW // TB
DEPTH = 4
G0 = 128
G1 = HIST - G0
FULL_CHUNKS = HIST // L
TAIL = HIST - FULL_CHUNKS * L


def _sc_body(tok_hbm, table_hbm, out_hbm, tokbuf, rows, outbuf,
             tsem, gsem0, gsem1, gsem2, gsem3):
    wid = lax.axis_index("s") * NC + lax.axis_index("c")
    base = wid * RPW

    def issue_gathers(r_local, tslot, rslot, gsem):
        rl = r_local & (TB - 1)
        pltpu.async_copy(
            table_hbm.at[tokbuf.at[tslot, rl, pl.ds(0, G0)]],
            rows.at[rslot, pl.ds(0, G0)], gsem)
        pltpu.async_copy(
            table_hbm.at[tokbuf.at[tslot, rl, pl.ds(G0, G1)]],
            rows.at[rslot, pl.ds(G0, G1)], gsem)

    def wait_gathers(rslot, gsem):
        pltpu.make_async_copy(
            table_hbm.at[tokbuf.at[0, 0, pl.ds(0, G0)]],
            rows.at[rslot, pl.ds(0, G0)], gsem).wait()
        pltpu.make_async_copy(
            table_hbm.at[tokbuf.at[0, 0, pl.ds(G0, G1)]],
            rows.at[rslot, pl.ds(G0, G1)], gsem).wait()

    def compute_row(r_local, tslot, rslot):
        rl = r_local & (TB - 1)
        zeros = jnp.zeros((L,), jnp.float32)
        izeros = jnp.zeros((L,), jnp.int32)

        def chunk(i, carry):
            a0, a1, a2, a3, cnt = carry
            t = tokbuf[tslot, rl, pl.ds(i * L, L)]
            cnt = cnt + plsc.all_reduce_population_count(t != 0)
            for k in range(L):
                rr = i * L + k
                a0 = a0 + rows[rslot, rr, pl.ds(0, L)]
                a1 = a1 + rows[rslot, rr, pl.ds(L, L)]
                a2 = a2 + rows[rslot, rr, pl.ds(2 * L, L)]
                a3 = a3 + rows[rslot, rr, pl.ds(3 * L, L)]
            return (a0, a1, a2, a3, cnt)

        a0, a1, a2, a3, cnt = lax.fori_loop(
            0, FULL_CHUNKS, chunk, (zeros, zeros, zeros, zeros, izeros))

        t = tokbuf[tslot, rl, pl.ds(HIST - L, L)]
        lane = lax.iota(jnp.int32, 16)
        cnt = cnt + plsc.all_reduce_population_count(
            (lane >= L - TAIL) & (t != 0))
        for k in range(TAIL):
            rr = FULL_CHUNKS * L + k
            a0 = a0 + rows[rslot, rr, pl.ds(0, L)]
            a1 = a1 + rows[rslot, rr, pl.ds(L, L)]
            a2 = a2 + rows[rslot, rr, pl.ds(2 * L, L)]
            a3 = a3 + rows[rslot, rr, pl.ds(3 * L, L)]

        sv = 1.0 / jnp.maximum(cnt.astype(jnp.float32), 1.0)
        outbuf[r_local, pl.ds(0, L)] = a0 * sv
        outbuf[r_local, pl.ds(L, L)] = a1 * sv
        outbuf[r_local, pl.ds(2 * L, L)] = a2 * sv
        outbuf[r_local, pl.ds(3 * L, L)] = a3 * sv

    gsems = (gsem0, gsem1, gsem2, gsem3)

    pltpu.sync_copy(tok_hbm.at[pl.ds(base, TB)], tokbuf.at[0])
    for i in range(DEPTH - 1):
        issue_gathers(i, 0, i, gsems[i])

    GPB = TB // DEPTH
    NG = RPW // DEPTH

    def group(g, _):
        b = g // GPB
        tslot = b & 1
        r = DEPTH * g

        @pl.when(jnp.logical_and((g & (GPB - 1)) == 0, b + 1 < NBLK))
        def _():
            pltpu.async_copy(
                tok_hbm.at[pl.ds(base + (b + 1) * TB, TB)],
                tokbuf.at[(b + 1) & 1], tsem)

        @pl.when(jnp.logical_and((g & (GPB - 1)) == GPB - 1, b + 1 < NBLK))
        def _():
            pltpu.make_async_copy(
                tok_hbm.at[pl.ds(base, TB)], tokbuf.at[0], tsem).wait()

        for s in range(DEPTH):
            rr = r + s
            nxt = rr + DEPTH - 1
            nslot = (s + DEPTH - 1) % DEPTH

            wait_gathers(s, gsems[s])

            if s == 0:
                issue_gathers(nxt, (nxt // TB) & 1, nslot, gsems[nslot])
            else:
                @pl.when(g < NG - 1)
                def _():
                    issue_gathers(nxt, (nxt // TB) & 1, nslot, gsems[nslot])

            compute_row(rr, tslot, s)
        return ()

    lax.fori_loop(0, NG, group, ())
    pltpu.sync_copy(outbuf, out_hbm.at[pl.ds(base, RPW)])


_sc_kernel = functools.partial(
    pl.kernel,
    out_type=jax.ShapeDtypeStruct((BATCH, D), jnp.float32),
    mesh=plsc.VectorSubcoreMesh(
        core_axis_name="c", subcore_axis_name="s",
        num_cores=NC, num_subcores=NS),
    scratch_types=[
        pltpu.VMEM((2, TB, HIST), jnp.int32),
        pltpu.VMEM((DEPTH, HIST, D), jnp.float32),
        pltpu.VMEM((RPW, D), jnp.float32),
        pltpu.SemaphoreType.DMA,
        pltpu.SemaphoreType.DMA,
        pltpu.SemaphoreType.DMA,
        pltpu.SemaphoreType.DMA,
        pltpu.SemaphoreType.DMA,
    ],
    compiler_params=pltpu.CompilerParams(
        needs_layout_passes=False, use_tc_tiling_on_sc=False),
)(_sc_body)


def kernel(token_ids, table):
    return _sc_kernel(token_ids, table)
